# Initial kernel scaffold; baseline (speedup 1.0000x reference)
import jax
import jax.numpy as jnp
from jax.experimental import pallas as pl


def kernel(x_v, x_a, arc_index, W_vv, b_vv, W_va, W_att, b_att, W_aa, b_aa, W_av):
    raise NotImplementedError("write your pallas kernel here")


# speedup vs baseline: 10.6135x; 10.6135x over previous
import jax
import jax.numpy as jnp
from jax import lax
from jax.experimental import pallas as pl
from jax.experimental.pallas import tpu as pltpu
from jax.experimental.pallas import tpu_sc as plsc

F32 = jnp.float32

_CH = 80
_RB = 2000
_RCH = _RB // _CH


def _edge_pre_body(xa_ref, w8e_ref, r8_ref, w8aa_ref, b8aa_ref,
                   y_ref, xr_ref, z_ref):
    xa = xa_ref[...]
    e8 = jnp.dot(xa, w8e_ref[...], preferred_element_type=F32)
    ex = jnp.exp(e8)
    xr = jnp.dot(ex, r8_ref[...], preferred_element_type=F32)
    xr_ref[...] = xr
    y_ref[...] = xr * xa
    z_ref[...] = jnp.dot(xa, w8aa_ref[...], preferred_element_type=F32) + b8aa_ref[...]


def _node_pre_body(xv_ref, wvvt_ref, bvv_ref, wavt_ref, hself_ref, p_ref):
    xv = xv_ref[...]
    hself_ref[...] = jnp.dot(xv, wvvt_ref[...], preferred_element_type=F32) + bvv_ref[...]
    p_ref[...] = jnp.dot(xv, wavt_ref[...], preferred_element_type=F32)


def _finalize_body(t_ref, d_ref, hself_ref, wvat_ref, hv_ref):
    t = t_ref[0] + t_ref[1]
    den = d_ref[0] + d_ref[1]
    s = jnp.where(den > 0.0, t / den, 0.0)
    hv_ref[...] = hself_ref[...] + jnp.dot(s, wvat_ref[...], preferred_element_type=F32)


def _sc_body(ns, npt, nchunk, epw,
             y_h, xr_h, z_h, p_h, dst_h, src_h, zrow_h,
             t_out, d_out, ha_out,
             idxd, idxs, ybuf, xrbuf, zb, pa, pb, wbuf, tsh, dsh, sem):
    c = lax.axis_index("c")
    s = lax.axis_index("s")
    w = c * ns + s
    e0w = w * epw

    pltpu.sync_copy(dst_h.at[w], idxd)
    pltpu.sync_copy(src_h.at[w], idxs)

    pltpu.sync_copy(zrow_h, wbuf)
    pltpu.sync_copy(wbuf, tsh.at[pl.ds(s * npt, npt), :])
    pltpu.sync_copy(wbuf, dsh.at[pl.ds(s * npt, npt), :])
    plsc.subcore_barrier()

    def p1(j, carry):
        @pl.when(j % _RCH == 0)
        def _refill():
            blk = j // _RCH
            pltpu.sync_copy(y_h.at[pl.ds(e0w + blk * _RB, _RB), :], ybuf)
            pltpu.sync_copy(xr_h.at[pl.ds(e0w + blk * _RB, _RB), :], xrbuf)
        jj = j % _RCH
        pltpu.sync_copy(ybuf.at[pl.ds(jj * _CH, _CH), :], tsh.at[idxd.at[j]], add=True)
        pltpu.sync_copy(xrbuf.at[pl.ds(jj * _CH, _CH), :], dsh.at[idxd.at[j]], add=True)
        return carry

    lax.fori_loop(0, nchunk, p1, 0)
    plsc.subcore_barrier()

    pltpu.sync_copy(tsh.at[pl.ds(s * npt, npt), :], wbuf)
    pltpu.sync_copy(wbuf, t_out.at[c, pl.ds(s * npt, npt), :])
    pltpu.sync_copy(dsh.at[pl.ds(s * npt, npt), :], wbuf)
    pltpu.sync_copy(wbuf, d_out.at[c, pl.ds(s * npt, npt), :])

    def p2(j, carry):
        e0 = e0w + j * _CH
        pltpu.sync_copy(z_h.at[pl.ds(e0, _CH), :], zb)
        ca = pltpu.async_copy(p_h.at[idxs.at[j]], pa, sem)
        cb = pltpu.async_copy(p_h.at[idxd.at[j]], pb, sem)
        ca.wait()
        cb.wait()
        for r in range(_CH):
            zb[r, :] = zb[r, :] + pa[r, :] + pb[r, :]
        pltpu.sync_copy(zb, ha_out.at[pl.ds(e0, _CH), :])
        return carry

    lax.fori_loop(0, nchunk, p2, 0)


def kernel(x_v, x_a, arc_index, W_vv, b_vv, W_va, W_att, b_att, W_aa, b_aa, W_av):
    N, IN_V = x_v.shape
    E, IN_A = x_a.shape
    OUT_V = W_vv.shape[0]
    E8 = E // 8

    eye8 = jnp.eye(8, dtype=F32)
    w8e = jnp.kron(eye8, W_att.T)
    r8 = jnp.kron(eye8, jnp.ones((1, IN_A), F32))
    w8aa = jnp.kron(eye8, W_aa.T)
    b8aa = jnp.tile(b_aa, 8)[None, :]
    xa8 = x_a.reshape(E8, 8 * IN_A)

    BE = 2000
    grid_e = E8 // BE
    y8, xr8, z8 = pl.pallas_call(
        _edge_pre_body,
        grid=(grid_e,),
        in_specs=[
            pl.BlockSpec((BE, 8 * IN_A), lambda i: (i, 0)),
            pl.BlockSpec((8 * IN_A, 8), lambda i: (0, 0)),
            pl.BlockSpec((8, 8 * IN_A), lambda i: (0, 0)),
            pl.BlockSpec((8 * IN_A, 8 * IN_A), lambda i: (0, 0)),
            pl.BlockSpec((1, 8 * IN_A), lambda i: (0, 0)),
        ],
        out_specs=[pl.BlockSpec((BE, 8 * IN_A), lambda i: (i, 0))] * 3,
        out_shape=[jax.ShapeDtypeStruct((E8, 8 * IN_A), F32)] * 3,
    )(xa8, w8e, r8, w8aa, b8aa)

    BN = 1000
    grid_n = N // BN
    hself, p = pl.pallas_call(
        _node_pre_body,
        grid=(grid_n,),
        in_specs=[
            pl.BlockSpec((BN, IN_V), lambda i: (i, 0)),
            pl.BlockSpec((IN_V, OUT_V), lambda i: (0, 0)),
            pl.BlockSpec((1, OUT_V), lambda i: (0, 0)),
            pl.BlockSpec((IN_V, IN_A), lambda i: (0, 0)),
        ],
        out_specs=[
            pl.BlockSpec((BN, OUT_V), lambda i: (i, 0)),
            pl.BlockSpec((BN, IN_A), lambda i: (i, 0)),
        ],
        out_shape=[
            jax.ShapeDtypeStruct((N, OUT_V), F32),
            jax.ShapeDtypeStruct((N, IN_A), F32),
        ],
    )(x_v, W_vv.T, b_vv[None, :], W_av.T)

    info = plsc.get_sparse_core_info()
    nc, ns = info.num_cores, info.num_subcores
    nw = nc * ns
    epw = E // nw
    nchunk = epw // _CH
    npt = -(-N // (8 * ns)) * 8
    npad = npt * ns

    mesh = plsc.VectorSubcoreMesh(core_axis_name="c", subcore_axis_name="s")
    sc = pl.kernel(
        lambda *refs: _sc_body(ns, npt, nchunk, epw, *refs),
        out_type=[
            jax.ShapeDtypeStruct((nc, npad, IN_A), F32),
            jax.ShapeDtypeStruct((nc, npad, IN_A), F32),
            jax.ShapeDtypeStruct((E, IN_A), F32),
        ],
        mesh=mesh,
        compiler_params=pltpu.CompilerParams(use_tc_tiling_on_sc=False),
        scratch_types=[
            pltpu.VMEM((nchunk, _CH), jnp.int32),
            pltpu.VMEM((nchunk, _CH), jnp.int32),
            pltpu.VMEM((_RB, IN_A), F32),
            pltpu.VMEM((_RB, IN_A), F32),
            pltpu.VMEM((_CH, IN_A), F32),
            pltpu.VMEM((_CH, IN_A), F32),
            pltpu.VMEM((_CH, IN_A), F32),
            pltpu.VMEM((npt, IN_A), F32),
            pltpu.VMEM_SHARED((npad, IN_A), F32),
            pltpu.VMEM_SHARED((npad, IN_A), F32),
            pltpu.SemaphoreType.DMA,
        ],
    )
    t_p, d_p, h_a = sc(
        y8.reshape(E, IN_A),
        xr8.reshape(E, IN_A),
        z8.reshape(E, IN_A),
        p,
        arc_index[1].reshape(nw, nchunk, _CH),
        arc_index[0].reshape(nw, nchunk, _CH),
        jnp.zeros((npt, IN_A), F32),
    )

    h_v = pl.pallas_call(
        _finalize_body,
        grid=(grid_n,),
        in_specs=[
            pl.BlockSpec((nc, BN, IN_A), lambda i: (0, i, 0)),
            pl.BlockSpec((nc, BN, IN_A), lambda i: (0, i, 0)),
            pl.BlockSpec((BN, OUT_V), lambda i: (i, 0)),
            pl.BlockSpec((IN_A, OUT_V), lambda i: (0, 0)),
        ],
        out_specs=pl.BlockSpec((BN, OUT_V), lambda i: (i, 0)),
        out_shape=jax.ShapeDtypeStruct((N, OUT_V), F32),
    )(t_p, d_p, hself, W_va.T)

    return (h_v, h_a)
